# Initial kernel scaffold; baseline (speedup 1.0000x reference)
import jax
import jax.numpy as jnp
from jax.experimental import pallas as pl


def kernel(x_job, x_m, edge_index_jj, edge_index_jm, edge_index_mj, edge_index_mm, edge_attr_jm, edge_attr_mj, params):
    raise NotImplementedError("write your pallas kernel here")


# speedup vs baseline: 3.6753x; 3.6753x over previous
import functools
import jax
import jax.numpy as jnp
from jax import lax
from jax.experimental import pallas as pl
from jax.experimental.pallas import tpu as pltpu
from jax.experimental.pallas import tpu_sc as plsc

NJ, NM, HID = 10000, 2000, 256
NJp, NMp = 10240, 2048
NC, NS, K = 2, 16, 128
EJJp, EJMp, EMJp, EMMp = 163840, 81920, 81920, 16384
F32 = jnp.float32


def _mesh():
    return plsc.VectorSubcoreMesh(core_axis_name="c", subcore_axis_name="s",
                                  num_cores=NC, num_subcores=NS)


def _zero_zbuf(zbuf, f):
    zr_rows = zbuf.shape[0]
    nseg = f // 16
    def zr(i, _):
        zbuf[i // nseg, pl.ds((i % nseg) * 16, 16)] = jnp.zeros((16,), F32)
        return 0
    lax.fori_loop(0, zr_rows * nseg, zr, 0)


def _zero_acc(zbuf, acc, vp, s):
    zr_rows = zbuf.shape[0]
    rt = vp // NS
    base = s * rt
    for j in range(rt // zr_rows):
        pltpu.sync_copy(zbuf, acc.at[pl.ds(base + j * zr_rows, zr_rows)])


def _gs_stream(tab, sr, dr, acc, sbuf, dbuf, rows, sem, c, s):
    ept = sr.shape[1] // NS
    def body(k_, _):
        base = s * ept + k_ * K
        pltpu.sync_copy(sr.at[c, pl.ds(base, K)], sbuf)
        pltpu.sync_copy(dr.at[c, pl.ds(base, K)], dbuf)
        pltpu.async_copy(tab.at[sbuf], rows, sem).wait()
        pltpu.sync_copy(rows, acc.at[dbuf], add=True)
        return 0
    lax.fori_loop(0, ept // K, body, 0)


def _attr_stream(ear, dr, acc, dbuf, rows, c, s):
    ept = ear.shape[1] // NS
    def body(k_, _):
        base = s * ept + k_ * K
        pltpu.sync_copy(dr.at[c, pl.ds(base, K)], dbuf)
        pltpu.sync_copy(ear.at[c, pl.ds(base, K)], rows)
        pltpu.sync_copy(rows, acc.at[dbuf], add=True)
        return 0
    lax.fori_loop(0, ept // K, body, 0)


def _readout(acc, out, vp, c, s):
    rt = vp // NS
    b = s * rt
    pltpu.sync_copy(acc.at[pl.ds(b, rt)], out.at[c, pl.ds(b, rt)])


def _sc_agg_pair(tab, s1, d1, s2, d2, vd1p, vd2p):
    @functools.partial(
        pl.kernel,
        out_type=[jax.ShapeDtypeStruct((NC, vd1p, 128), F32),
                  jax.ShapeDtypeStruct((NC, vd2p, 128), F32)],
        mesh=_mesh(),
        scratch_types=[
            pltpu.VMEM_SHARED((vd1p, 128), F32),
            pltpu.VMEM_SHARED((vd2p, 128), F32),
            pltpu.VMEM((K,), jnp.int32),
            pltpu.VMEM((K,), jnp.int32),
            pltpu.VMEM((K, 128), F32),
            pltpu.VMEM((64, 128), F32),
            pltpu.SemaphoreType.DMA,
        ],
    )
    def k(tab_r, s1r, d1r, s2r, d2r, o1, o2, acc1, acc2, sbuf, dbuf, rows,
          zbuf, sem):
        c = lax.axis_index("c")
        s = lax.axis_index("s")
        _zero_zbuf(zbuf, 128)
        _zero_acc(zbuf, acc1, vd1p, s)
        _zero_acc(zbuf, acc2, vd2p, s)
        plsc.subcore_barrier()
        _gs_stream(tab_r, s1r, d1r, acc1, sbuf, dbuf, rows, sem, c, s)
        _gs_stream(tab_r, s2r, d2r, acc2, sbuf, dbuf, rows, sem, c, s)
        plsc.subcore_barrier()
        _readout(acc1, o1, vd1p, c, s)
        _readout(acc2, o2, vd2p, c, s)

    return k(tab, s1, d1, s2, d2)


def _sc_agg_l1(xj16, xm16, sJJ, dJJ, sJM, dJM, eaJM, sMJ, dMJ, eaMJ, sMM, dMM):
    @functools.partial(
        pl.kernel,
        out_type=[jax.ShapeDtypeStruct((NC, NJp, 16), F32),
                  jax.ShapeDtypeStruct((NC, NJp, 16), F32),
                  jax.ShapeDtypeStruct((NC, NJp, 16), F32),
                  jax.ShapeDtypeStruct((NC, NMp, 16), F32),
                  jax.ShapeDtypeStruct((NC, NMp, 16), F32),
                  jax.ShapeDtypeStruct((NC, NMp, 16), F32)],
        mesh=_mesh(),
        compiler_params=pltpu.CompilerParams(use_tc_tiling_on_sc=False),
        scratch_types=[
            pltpu.VMEM_SHARED((NJp, 16), F32),
            pltpu.VMEM_SHARED((NJp, 16), F32),
            pltpu.VMEM_SHARED((NJp, 16), F32),
            pltpu.VMEM_SHARED((NMp, 16), F32),
            pltpu.VMEM_SHARED((NMp, 16), F32),
            pltpu.VMEM_SHARED((NMp, 16), F32),
            pltpu.VMEM((K,), jnp.int32),
            pltpu.VMEM((K,), jnp.int32),
            pltpu.VMEM((K, 16), F32),
            pltpu.VMEM((128, 16), F32),
            pltpu.SemaphoreType.DMA,
        ],
    )
    def k(xj_r, xm_r, sJJr, dJJr, sJMr, dJMr, eaJMr, sMJr, dMJr, eaMJr,
          sMMr, dMMr, oJJ, oMJ, oAJ, oJM, oMM, oAM,
          aJJ, aMJ, aAJ, aJM, aMM, aAM, sbuf, dbuf, rows, zbuf, sem):
        c = lax.axis_index("c")
        s = lax.axis_index("s")
        _zero_zbuf(zbuf, 16)
        for acc, vp in ((aJJ, NJp), (aMJ, NJp), (aAJ, NJp),
                        (aJM, NMp), (aMM, NMp), (aAM, NMp)):
            _zero_acc(zbuf, acc, vp, s)
        plsc.subcore_barrier()
        _gs_stream(xj_r, sJJr, dJJr, aJJ, sbuf, dbuf, rows, sem, c, s)
        _gs_stream(xj_r, sJMr, dJMr, aJM, sbuf, dbuf, rows, sem, c, s)
        _attr_stream(eaJMr, dJMr, aAM, dbuf, rows, c, s)
        _gs_stream(xm_r, sMJr, dMJr, aMJ, sbuf, dbuf, rows, sem, c, s)
        _attr_stream(eaMJr, dMJr, aAJ, dbuf, rows, c, s)
        _gs_stream(xm_r, sMMr, dMMr, aMM, sbuf, dbuf, rows, sem, c, s)
        plsc.subcore_barrier()
        for acc, out, vp in ((aJJ, oJJ, NJp), (aMJ, oMJ, NJp), (aAJ, oAJ, NJp),
                             (aJM, oJM, NMp), (aMM, oMM, NMp), (aAM, oAM, NMp)):
            _readout(acc, out, vp, c, s)

    return k(xj16, xm16, sJJ, dJJ, sJM, dJM, eaJM, sMJ, dMJ, eaMJ, sMM, dMM)


def _dot(a, b):
    return jnp.dot(a, b, preferred_element_type=F32)


def _mt_transform(x_m, w, b):
    def body(x_ref, w_ref, b_ref, o_ref):
        y = _dot(x_ref[...], w_ref[...]) + b_ref[...]
        o_ref[...] = jnp.concatenate([y, jnp.zeros_like(y)], axis=1)

    return pl.pallas_call(
        body,
        out_shape=jax.ShapeDtypeStruct((NM, 16), F32),
    )(x_m, w, b.reshape(1, 8))


def _full(shape):
    return pl.BlockSpec(shape, lambda i: tuple(0 for _ in shape))


def _mlp_l1(aggA, aggB, attrP, pA, pB, vp, bm=512):
    w1A, b1A = pA['l1']['W'], pA['l1']['b'].reshape(1, HID)
    w2A, b2A = pA['l2']['W'], pA['l2']['b'].reshape(1, HID)
    w1B = pB['l1']['W']
    w1Ba, w1Be = w1B[:8], w1B[8:9]
    b1B = pB['l1']['b'].reshape(1, HID)
    w2B, b2B = pB['l2']['W'], pB['l2']['b'].reshape(1, HID)

    def body(aA_ref, aB_ref, at_ref, w1A_r, b1A_r, w2A_r, b2A_r,
             w1Ba_r, w1Be_r, b1B_r, w2B_r, b2B_r, o_ref):
        a = aA_ref[0, :, :8] + aA_ref[1, :, :8]
        bb = aB_ref[0, :, :8] + aB_ref[1, :, :8]
        sa = at_ref[0, :, 0:1] + at_ref[1, :, 0:1]
        hA = jnp.maximum(_dot(a, w1A_r[...]) + b1A_r[...], 0.)
        oA = _dot(hA, w2A_r[...]) + b2A_r[...]
        hB = jnp.maximum(_dot(bb, w1Ba_r[...]) + sa * w1Be_r[...] + b1B_r[...], 0.)
        oB = _dot(hB, w2B_r[...]) + b2B_r[...]
        x = jnp.maximum(oA + oB, 0.)
        o_ref[0] = x[:, :128]
        o_ref[1] = x[:, 128:]

    return pl.pallas_call(
        body,
        grid=(vp // bm,),
        in_specs=[pl.BlockSpec((NC, bm, 16), lambda i: (0, i, 0)),
                  pl.BlockSpec((NC, bm, 16), lambda i: (0, i, 0)),
                  pl.BlockSpec((NC, bm, 16), lambda i: (0, i, 0)),
                  _full((8, HID)), _full((1, HID)), _full((HID, HID)),
                  _full((1, HID)), _full((8, HID)), _full((1, HID)),
                  _full((1, HID)), _full((HID, HID)), _full((1, HID))],
        out_specs=pl.BlockSpec((NC, bm, 128), lambda i: (0, i, 0)),
        out_shape=jax.ShapeDtypeStruct((NC, vp, 128), F32),
    )(aggA, aggB, attrP, w1A, b1A, w2A, b2A, w1Ba, w1Be, b1B, w2B, b2B)


def _mlp_l23(yA, yB, attrP, pA, pB, vp, final, n_real, bm=512):
    w1A, b1A = pA['l1']['W'], pA['l1']['b'].reshape(1, HID)
    w2A, b2A = pA['l2']['W'], pA['l2']['b'].reshape(1, HID)
    w1B = pB['l1']['W']
    w1Ba, w1Be = w1B[:HID], w1B[HID:HID + 1]
    b1B = pB['l1']['b'].reshape(1, HID)
    w2B, b2B = pB['l2']['W'], pB['l2']['b'].reshape(1, HID)

    def compute_x(yA_ref, yB_ref, at_ref, w1A_r, w2A_r, w1Ba_r, w1Be_r,
                  w2B_r, b1A_r, b2A_r, b1B_r, b2B_r):
        sa = at_ref[0, :, 0:1] + at_ref[1, :, 0:1]
        hA = jnp.maximum(_dot(yA_ref[0], w1A_r[:128]) +
                         _dot(yA_ref[1], w1A_r[128:]) + b1A_r[...], 0.)
        oA = _dot(hA, w2A_r[...]) + b2A_r[...]
        hB = jnp.maximum(_dot(yB_ref[0], w1Ba_r[:128]) +
                         _dot(yB_ref[1], w1Ba_r[128:]) +
                         sa * w1Be_r[...] + b1B_r[...], 0.)
        oB = _dot(hB, w2B_r[...]) + b2B_r[...]
        return jnp.maximum(oA + oB, 0.)

    if final is None:
        def body(yA_ref, yB_ref, at_ref, w1A_r, b1A_r, w2A_r, b2A_r, w1Ba_r,
                 w1Be_r, b1B_r, w2B_r, b2B_r, o_ref):
            x = compute_x(yA_ref, yB_ref, at_ref, w1A_r, w2A_r, w1Ba_r,
                          w1Be_r, w2B_r, b1A_r, b2A_r, b1B_r, b2B_r)
            o_ref[0] = x[:, :128]
            o_ref[1] = x[:, 128:]

        return pl.pallas_call(
            body,
            grid=(vp // bm,),
            in_specs=[pl.BlockSpec((NC, bm, 128), lambda i: (0, i, 0)),
                      pl.BlockSpec((NC, bm, 128), lambda i: (0, i, 0)),
                      pl.BlockSpec((NC, bm, 16), lambda i: (0, i, 0)),
                      _full((HID, HID)), _full((1, HID)), _full((HID, HID)),
                      _full((1, HID)), _full((HID, HID)), _full((1, HID)),
                      _full((1, HID)), _full((HID, HID)), _full((1, HID))],
            out_specs=pl.BlockSpec((NC, bm, 128), lambda i: (0, i, 0)),
            out_shape=jax.ShapeDtypeStruct((NC, vp, 128), F32),
        )(yA, yB, attrP, w1A, b1A, w2A, b2A, w1Ba, w1Be, b1B, w2B, b2B)

    wf, bf = final['W'], final['b'].reshape(1, HID)

    def body(yA_ref, yB_ref, at_ref, w1A_r, b1A_r, w2A_r, b2A_r, w1Ba_r,
             w1Be_r, b1B_r, w2B_r, b2B_r, wf_r, bf_r, o_ref, osum_ref):
        i = pl.program_id(0)
        x = compute_x(yA_ref, yB_ref, at_ref, w1A_r, w2A_r, w1Ba_r,
                      w1Be_r, w2B_r, b1A_r, b2A_r, b1B_r, b2B_r)
        xo = _dot(x, wf_r[...]) + bf_r[...]
        o_ref[...] = xo
        rows = i * bm + lax.broadcasted_iota(jnp.int32, (bm, 1), 0)
        xm = jnp.where(rows < n_real, xo, 0.)

        @pl.when(i == 0)
        def _():
            osum_ref[...] = jnp.zeros_like(osum_ref)

        osum_ref[...] += jnp.sum(xm, axis=0, keepdims=True)

    return pl.pallas_call(
        body,
        grid=(vp // bm,),
        in_specs=[pl.BlockSpec((NC, bm, 128), lambda i: (0, i, 0)),
                  pl.BlockSpec((NC, bm, 128), lambda i: (0, i, 0)),
                  pl.BlockSpec((NC, bm, 16), lambda i: (0, i, 0)),
                  _full((HID, HID)), _full((1, HID)), _full((HID, HID)),
                  _full((1, HID)), _full((HID, HID)), _full((1, HID)),
                  _full((1, HID)), _full((HID, HID)), _full((1, HID)),
                  _full((HID, HID)), _full((1, HID))],
        out_specs=[pl.BlockSpec((bm, HID), lambda i: (i, 0)),
                   pl.BlockSpec((1, HID), lambda i: (0, 0))],
        out_shape=[jax.ShapeDtypeStruct((vp, HID), F32),
                   jax.ShapeDtypeStruct((1, HID), F32)],
    )(yA, yB, attrP, w1A, b1A, w2A, b2A, w1Ba, w1Be, b1B, w2B, b2B, wf, bf)


def _pad_edges(ei, epad, vdst):
    e = ei.shape[1]
    ar = jnp.arange(epad - e, dtype=jnp.int32)
    src = jnp.concatenate([ei[0], ar % 16])
    dst = jnp.concatenate([ei[1], vdst + (ar % 8)])
    return src, dst


@jax.jit
def _impl(x_job, x_m, ei_jj, ei_jm, ei_mj, ei_mm, ea_jm, ea_mj, params):
    sJJ, dJJ = _pad_edges(ei_jj, EJJp, NJ)
    sJM, dJM = _pad_edges(ei_jm, EJMp, NM)
    sMJ, dMJ = _pad_edges(ei_mj, EMJp, NJ)
    sMM, dMM = _pad_edges(ei_mm, EMMp, NM)

    r2 = lambda a: a.reshape(NC, -1)
    ea16_jm = jnp.zeros((EJMp, 16), F32).at[:ei_jm.shape[1], 0].set(ea_jm[:, 0])
    ea16_mj = jnp.zeros((EMJp, 16), F32).at[:ei_mj.shape[1], 0].set(ea_mj[:, 0])

    xj16 = jnp.pad(x_job, ((0, 0), (0, 8)))
    xm16 = _mt_transform(x_m, params['mt']['W'], params['mt']['b'])

    (aggJJ, aggMJ, attrJ, aggJM, aggMM, attrM) = _sc_agg_l1(
        xj16, xm16,
        r2(sJJ), r2(dJJ), r2(sJM), r2(dJM), ea16_jm.reshape(NC, -1, 16),
        r2(sMJ), r2(dMJ), ea16_mj.reshape(NC, -1, 16), r2(sMM), r2(dMM))

    L = params['layers']
    xj = _mlp_l1(aggJJ, aggMJ, attrJ, L[0]['nn1'], L[0]['nn3'], NJp)
    xm = _mlp_l1(aggMM, aggJM, attrM, L[0]['nn4'], L[0]['nn2'], NMp)

    st = lambda s_, off: jnp.stack([s_, s_ + off])
    dup = lambda d_: jnp.stack([d_, d_])
    sJJ2, dJJ2 = st(sJJ, NJp), dup(dJJ)
    sJM2, dJM2 = st(sJM, NJp), dup(dJM)
    sMJ2, dMJ2 = st(sMJ, NMp), dup(dMJ)
    sMM2, dMM2 = st(sMM, NMp), dup(dMM)

    for li in (1, 2):
        yJJ, yJM = _sc_agg_pair(xj.reshape(-1, 128), sJJ2, dJJ2, sJM2, dJM2,
                                NJp, NMp)
        yMJ, yMM = _sc_agg_pair(xm.reshape(-1, 128), sMJ2, dMJ2, sMM2, dMM2,
                                NJp, NMp)
        if li < 2:
            xj = _mlp_l23(yJJ, yMJ, attrJ, L[li]['nn1'], L[li]['nn3'], NJp,
                          None, NJ)
            xm = _mlp_l23(yMM, yJM, attrM, L[li]['nn4'], L[li]['nn2'], NMp,
                          None, NM)
        else:
            xjf, sumJ = _mlp_l23(yJJ, yMJ, attrJ, L[li]['nn1'], L[li]['nn3'],
                                 NJp, params['jfc'], NJ)
            xmf, sumM = _mlp_l23(yMM, yJM, attrM, L[li]['nn4'], L[li]['nn2'],
                                 NMp, params['mfc'], NM)

    x_graph = jnp.concatenate([sumJ, sumM], axis=1)
    return (x_graph, xjf[:NJ], xmf[:NM])


def kernel(x_job, x_m, edge_index_jj, edge_index_jm, edge_index_mj,
           edge_index_mm, edge_attr_jm, edge_attr_mj, params):
    return _impl(x_job, x_m, edge_index_jj, edge_index_jm, edge_index_mj,
                 edge_index_mm, edge_attr_jm, edge_attr_mj, params)
